# Initial kernel scaffold; baseline (speedup 1.0000x reference)
import jax
import jax.numpy as jnp
from jax.experimental import pallas as pl


def kernel(b_word_ids, b_char_ids, word_table, char_table, conv_w, conv_b):
    raise NotImplementedError("write your pallas kernel here")


# speedup vs baseline: 11.7646x; 11.7646x over previous
import functools

import jax
import jax.numpy as jnp
from jax import lax
from jax.experimental import pallas as pl
from jax.experimental.pallas import tpu as pltpu
from jax.experimental.pallas import tpu_sc as plsc

NC, NS = 2, 16
NW = NC * NS


def _sc_gather(wids, cids, word_table, char_table, T, L, WD, CD):
    TW = T // NW
    CT = 128
    NSUB = (CT * L) // 128
    assert TW % CT == 0 and (CT * L) % 128 == 0

    mesh = plsc.VectorSubcoreMesh(core_axis_name="c", subcore_axis_name="s")

    @functools.partial(
        pl.kernel,
        mesh=mesh,
        compiler_params=pltpu.CompilerParams(use_tc_tiling_on_sc=False),
        out_type=(
            jax.ShapeDtypeStruct((T, WD), jnp.float32),
            jax.ShapeDtypeStruct((T * L, CD), jnp.float32),
        ),
        scratch_types=[
            pltpu.VMEM((CT,), jnp.int32),
            pltpu.VMEM((CT * L,), jnp.int32),
            pltpu.VMEM((CT, WD), jnp.float32),
            pltpu.VMEM((CT * L, CD), jnp.float32),
            pltpu.SemaphoreType.DMA,
            pltpu.SemaphoreType.DMA,
        ],
    )
    def sck(wids_h, cids_h, wt_h, ct_h, wout_h, ceout_h,
            widv, cidv, wrows, cerows, s1, s2):
        wid = lax.axis_index("s") * NC + lax.axis_index("c")

        def body(c, carry):
            base = wid * TW + c * CT
            pltpu.sync_copy(wids_h.at[pl.ds(base, CT)], widv)
            pltpu.sync_copy(cids_h.at[pl.ds(base * L, CT * L)], cidv)
            cw = pltpu.async_copy(wt_h.at[widv], wrows, s1)
            chs = []
            for j in range(NSUB):
                chs.append(pltpu.async_copy(
                    ct_h.at[cidv.at[pl.ds(j * 128, 128)]],
                    cerows.at[pl.ds(j * 128, 128)], s2))
            cw.wait()
            for h in chs:
                h.wait()
            pltpu.sync_copy(wrows, wout_h.at[pl.ds(base, CT)])
            pltpu.sync_copy(cerows, ceout_h.at[pl.ds(base * L, CT * L)])
            return carry

        lax.fori_loop(0, TW // CT, body, 0)

    return sck(wids, cids, word_table, char_table)


def _tc_conv_assemble(ce2, word_raw, wbig, wids_col, bias_row, T, L, WD, CV):
    TB = 1024
    assert T % TB == 0
    KD = ce2.shape[1]
    ND = wbig.shape[1]

    def tck(ce_ref, w_ref, wb_ref, ids_ref, b_ref, out_ref):
        y = jnp.dot(ce_ref[...].astype(jnp.bfloat16), wb_ref[...],
                    preferred_element_type=jnp.float32)
        y = y + b_ref[...]
        m = y[:, 0:CV]
        for p in range(1, L):
            m = jnp.maximum(m, y[:, p * CV:(p + 1) * CV])
        mask = (ids_ref[...] != 0).astype(jnp.float32)
        out_ref[:, :WD] = w_ref[...] * mask
        out_ref[:, WD:] = m

    return pl.pallas_call(
        tck,
        grid=(T // TB,),
        in_specs=[
            pl.BlockSpec((TB, KD), lambda i: (i, 0)),
            pl.BlockSpec((TB, WD), lambda i: (i, 0)),
            pl.BlockSpec((KD, ND), lambda i: (0, 0)),
            pl.BlockSpec((TB, 1), lambda i: (i, 0)),
            pl.BlockSpec((1, ND), lambda i: (0, 0)),
        ],
        out_specs=pl.BlockSpec((TB, WD + CV), lambda i: (i, 0)),
        out_shape=jax.ShapeDtypeStruct((T, WD + CV), jnp.float32),
    )(ce2, word_raw, wbig, wids_col, bias_row)


def kernel(b_word_ids, b_char_ids, word_table, char_table, conv_w, conv_b):
    B, S = b_word_ids.shape
    L = b_char_ids.shape[2]
    WD = word_table.shape[1]
    CD = char_table.shape[1]
    CV = conv_w.shape[0]
    T = B * S

    wids = b_word_ids.reshape(T)
    cids = b_char_ids.reshape(T * L)
    ct0 = char_table.at[0].set(0.0)

    word_raw, ce = _sc_gather(wids, cids, word_table, ct0, T, L, WD, CD)

    W4 = jnp.zeros((L, CD, L, CV), jnp.float32)
    for p in range(L):
        for k in range(5):
            l = p + k - 2
            if 0 <= l < L:
                W4 = W4.at[l, :, p, :].set(conv_w[:, :, k].T)
    wbig = W4.reshape(L * CD, L * CV).astype(jnp.bfloat16)
    bias_row = jnp.tile(conv_b, L)[None, :]

    ce2 = ce.reshape(T, L * CD)
    out = _tc_conv_assemble(ce2, word_raw, wbig, wids.reshape(T, 1),
                            bias_row, T, L, WD, CV)
    return out.reshape(B, S, WD + CV)
